# Initial kernel scaffold; baseline (speedup 1.0000x reference)
import jax
import jax.numpy as jnp
from jax.experimental import pallas as pl


def kernel(queries, memory_bank):
    raise NotImplementedError("write your pallas kernel here")


# speedup vs baseline: 3.1663x; 3.1663x over previous
import jax
import jax.numpy as jnp
from jax.experimental import pallas as pl

B, H, W, D, K = 8, 28, 28, 384, 20000
QB = H * W
KB = 2048
K_PAD = ((K + KB - 1) // KB) * KB
NK = K_PAD // KB
PAD_VAL = 1e4


def _knn_kernel(q_ref, m_ref, patch_ref, img_ref):
    k = pl.program_id(1)
    q = q_ref[...]
    m = m_ref[...]
    m_sq = jnp.sum(m * m, axis=0)
    dot = jax.lax.dot_general(
        q, m, dimension_numbers=(((1,), (0,)), ((), ())),
        preferred_element_type=jnp.float32)
    t = m_sq[None, :] - 2.0 * dot
    blk_min = jnp.min(t, axis=1)

    @pl.when(k == 0)
    def _init():
        patch_ref[0, 0, :] = blk_min

    @pl.when(k > 0)
    def _acc():
        patch_ref[0, 0, :] = jnp.minimum(patch_ref[0, 0, :], blk_min)

    @pl.when(k == NK - 1)
    def _finish():
        q_sq = jnp.sum(q * q, axis=1)
        d2 = jnp.maximum(q_sq + patch_ref[0, 0, :], 1e-12)
        nn = jnp.sqrt(d2)
        patch_ref[0, 0, :] = nn
        img_ref[0, 0, :] = jnp.full((128,), jnp.max(nn), dtype=jnp.float32)


@jax.jit
def kernel(queries, memory_bank):
    mem_t = jnp.pad(memory_bank, ((0, K_PAD - K), (0, 0)),
                    constant_values=PAD_VAL).T
    patch, img = pl.pallas_call(
        _knn_kernel,
        grid=(B, NK),
        in_specs=[
            pl.BlockSpec((QB, D), lambda i, k: (i, 0)),
            pl.BlockSpec((D, KB), lambda i, k: (0, k)),
        ],
        out_specs=[
            pl.BlockSpec((1, 1, QB), lambda i, k: (i, 0, 0)),
            pl.BlockSpec((1, 1, 128), lambda i, k: (i, 0, 0)),
        ],
        out_shape=[
            jax.ShapeDtypeStruct((B, 1, QB), jnp.float32),
            jax.ShapeDtypeStruct((B, 1, 128), jnp.float32),
        ],
    )(queries.reshape(B * QB, D), mem_t)
    return patch.reshape(B, H, W), img[:, 0, 0]


# speedup vs baseline: 3.1897x; 1.0074x over previous
import jax
import jax.numpy as jnp
from jax.experimental import pallas as pl

B, H, W, D, K = 8, 28, 28, 384, 20000
QB = H * W
KB = 2048
K_PAD = ((K + KB - 1) // KB) * KB
NK = K_PAD // KB
PAD_VAL = 1e4


def _knn_kernel(q_ref, m_ref, patch_ref, img_ref):
    k = pl.program_id(1)
    q = q_ref[...]
    m = m_ref[...]
    m32 = m.astype(jnp.float32)
    m_sq = jnp.sum(m32 * m32, axis=0)
    dot = jax.lax.dot_general(
        q, m, dimension_numbers=(((1,), (0,)), ((), ())),
        preferred_element_type=jnp.float32)
    t = m_sq[None, :] - 2.0 * dot
    blk_min = jnp.min(t, axis=1)

    @pl.when(k == 0)
    def _init():
        patch_ref[0, 0, :] = blk_min

    @pl.when(k > 0)
    def _acc():
        patch_ref[0, 0, :] = jnp.minimum(patch_ref[0, 0, :], blk_min)

    @pl.when(k == NK - 1)
    def _finish():
        q32 = q.astype(jnp.float32)
        q_sq = jnp.sum(q32 * q32, axis=1)
        d2 = jnp.maximum(q_sq + patch_ref[0, 0, :], 1e-12)
        nn = jnp.sqrt(d2)
        patch_ref[0, 0, :] = nn
        img_ref[0, 0, :] = jnp.full((128,), jnp.max(nn), dtype=jnp.float32)


@jax.jit
def kernel(queries, memory_bank):
    mem_t = jnp.pad(memory_bank, ((0, K_PAD - K), (0, 0)),
                    constant_values=PAD_VAL).T.astype(jnp.bfloat16)
    patch, img = pl.pallas_call(
        _knn_kernel,
        grid=(B, NK),
        in_specs=[
            pl.BlockSpec((QB, D), lambda i, k: (i, 0)),
            pl.BlockSpec((D, KB), lambda i, k: (0, k)),
        ],
        out_specs=[
            pl.BlockSpec((1, 1, QB), lambda i, k: (i, 0, 0)),
            pl.BlockSpec((1, 1, 128), lambda i, k: (i, 0, 0)),
        ],
        out_shape=[
            jax.ShapeDtypeStruct((B, 1, QB), jnp.float32),
            jax.ShapeDtypeStruct((B, 1, 128), jnp.float32),
        ],
    )(queries.reshape(B * QB, D).astype(jnp.bfloat16), mem_t)
    return patch.reshape(B, H, W), img[:, 0, 0]


# speedup vs baseline: 3.2745x; 1.0266x over previous
import jax
import jax.numpy as jnp
from jax.experimental import pallas as pl
from jax.experimental.pallas import tpu as pltpu

B, H, W, D, K = 8, 28, 28, 384, 20000
QB = H * W
KB = 2048
K_PAD = ((K + KB - 1) // KB) * KB
NK = K_PAD // KB
D_AUG = 400
PAD_VAL = 1e4


def _knn_kernel(q_ref, m_ref, patch_ref, img_ref, acc_ref):
    k = pl.program_id(1)
    q = q_ref[...]
    m = m_ref[...]
    t = jax.lax.dot_general(
        q, m, dimension_numbers=(((1,), (0,)), ((), ())),
        preferred_element_type=jnp.float32)
    red = t[:, 0:128]
    for j in range(1, KB // 128):
        red = jnp.minimum(red, t[:, j * 128:(j + 1) * 128])

    @pl.when(k == 0)
    def _init():
        acc_ref[...] = red

    @pl.when(k > 0)
    def _acc():
        acc_ref[...] = jnp.minimum(acc_ref[...], red)

    @pl.when(k == NK - 1)
    def _finish():
        q32 = q[:, 0:D].astype(jnp.float32)
        q_sq = 0.25 * jnp.sum(q32 * q32, axis=1)
        d2 = jnp.maximum(q_sq + jnp.min(acc_ref[...], axis=1), 1e-12)
        nn = jnp.sqrt(d2)
        patch_ref[0, 0, :] = nn
        img_ref[0, 0, :] = jnp.full((128,), jnp.max(nn), dtype=jnp.float32)


@jax.jit
def kernel(queries, memory_bank):
    qn = queries.reshape(B * QB, D)
    q_aug = jnp.concatenate(
        [(-2.0 * qn).astype(jnp.bfloat16),
         jnp.ones((B * QB, 2), jnp.bfloat16),
         jnp.zeros((B * QB, D_AUG - D - 2), jnp.bfloat16)], axis=1)

    mem = jnp.pad(memory_bank, ((0, K_PAD - K), (0, 0)),
                  constant_values=PAD_VAL)
    m_sq = jnp.sum(mem * mem, axis=1)
    msq_hi = m_sq.astype(jnp.bfloat16)
    msq_lo = (m_sq - msq_hi.astype(jnp.float32)).astype(jnp.bfloat16)
    m_aug = jnp.concatenate(
        [mem.T.astype(jnp.bfloat16),
         msq_hi[None, :], msq_lo[None, :],
         jnp.zeros((D_AUG - D - 2, K_PAD), jnp.bfloat16)], axis=0)

    patch, img = pl.pallas_call(
        _knn_kernel,
        grid=(B, NK),
        in_specs=[
            pl.BlockSpec((QB, D_AUG), lambda i, k: (i, 0)),
            pl.BlockSpec((D_AUG, KB), lambda i, k: (0, k)),
        ],
        out_specs=[
            pl.BlockSpec((1, 1, QB), lambda i, k: (i, 0, 0)),
            pl.BlockSpec((1, 1, 128), lambda i, k: (i, 0, 0)),
        ],
        out_shape=[
            jax.ShapeDtypeStruct((B, 1, QB), jnp.float32),
            jax.ShapeDtypeStruct((B, 1, 128), jnp.float32),
        ],
        scratch_shapes=[pltpu.VMEM((QB, 128), jnp.float32)],
    )(q_aug, m_aug)
    return patch.reshape(B, H, W), img[:, 0, 0]


# speedup vs baseline: 3.4348x; 1.0490x over previous
import jax
import jax.numpy as jnp
from jax.experimental import pallas as pl
from jax.experimental.pallas import tpu as pltpu

B, H, W, D, K = 8, 28, 28, 384, 20000
QB = H * W
KB = 4096
CHUNK = 512
K_PAD = ((K + KB - 1) // KB) * KB
NK = K_PAD // KB
D_AUG = 400
PAD_VAL = 1e4


def _knn_kernel(q_ref, m_ref, patch_ref, img_ref, acc_ref):
    k = pl.program_id(1)
    q = q_ref[...]
    m = m_ref[...]
    red = None
    for c in range(KB // CHUNK):
        t = jax.lax.dot_general(
            q, m[:, c * CHUNK:(c + 1) * CHUNK],
            dimension_numbers=(((1,), (0,)), ((), ())),
            preferred_element_type=jnp.float32)
        for j in range(CHUNK // 128):
            sl = t[:, j * 128:(j + 1) * 128]
            red = sl if red is None else jnp.minimum(red, sl)

    @pl.when(k == 0)
    def _init():
        acc_ref[...] = red

    @pl.when(k > 0)
    def _acc():
        acc_ref[...] = jnp.minimum(acc_ref[...], red)

    @pl.when(k == NK - 1)
    def _finish():
        q32 = q[:, 0:D].astype(jnp.float32)
        q_sq = 0.25 * jnp.sum(q32 * q32, axis=1)
        d2 = jnp.maximum(q_sq + jnp.min(acc_ref[...], axis=1), 1e-12)
        nn = jnp.sqrt(d2)
        patch_ref[0, 0, :] = nn
        img_ref[0, 0, :] = jnp.full((128,), jnp.max(nn), dtype=jnp.float32)


@jax.jit
def kernel(queries, memory_bank):
    qn = queries.reshape(B * QB, D)
    q_aug = jnp.concatenate(
        [(-2.0 * qn).astype(jnp.bfloat16),
         jnp.ones((B * QB, 2), jnp.bfloat16),
         jnp.zeros((B * QB, D_AUG - D - 2), jnp.bfloat16)], axis=1)

    mem = jnp.pad(memory_bank, ((0, K_PAD - K), (0, 0)),
                  constant_values=PAD_VAL)
    m_sq = jnp.sum(mem * mem, axis=1)
    msq_hi = m_sq.astype(jnp.bfloat16)
    msq_lo = (m_sq - msq_hi.astype(jnp.float32)).astype(jnp.bfloat16)
    m_aug = jnp.concatenate(
        [mem.T.astype(jnp.bfloat16),
         msq_hi[None, :], msq_lo[None, :],
         jnp.zeros((D_AUG - D - 2, K_PAD), jnp.bfloat16)], axis=0)

    patch, img = pl.pallas_call(
        _knn_kernel,
        grid=(B, NK),
        in_specs=[
            pl.BlockSpec((QB, D_AUG), lambda i, k: (i, 0)),
            pl.BlockSpec((D_AUG, KB), lambda i, k: (0, k)),
        ],
        out_specs=[
            pl.BlockSpec((1, 1, QB), lambda i, k: (i, 0, 0)),
            pl.BlockSpec((1, 1, 128), lambda i, k: (i, 0, 0)),
        ],
        out_shape=[
            jax.ShapeDtypeStruct((B, 1, QB), jnp.float32),
            jax.ShapeDtypeStruct((B, 1, 128), jnp.float32),
        ],
        scratch_shapes=[pltpu.VMEM((QB, 128), jnp.float32)],
    )(q_aug, m_aug)
    return patch.reshape(B, H, W), img[:, 0, 0]
